# baseline (speedup 1.0000x reference)
import functools
import jax
import jax.numpy as jnp
import numpy as np
from jax import lax
from jax.experimental import pallas as pl
from jax.experimental.pallas import tpu as pltpu
from jax.experimental.pallas import tpu_sc as plsc

N_ATOMS = 10000
N_EDGES = 160000
N_TRIP = 160000
F = 128
N_RBF = 6
N_SPH = 7
N_SRAD = 7
N_BIL = 8
CUTOFF = 5.0
N_MOL = 100
N_INT = 7

NC, NS, L = 2, 16, 16
NW = NC * NS
EP = 163840
PER_W = EP // NW
NCHUNK = PER_W // 128
NAP = 10240
HALF_E = EP // 2
ACC_E = HALF_E + 128
BE = 1024
NBE = EP // BE
BA = 1024
NBA = NAP // BA
NMOLP = 104

_mesh = lambda: plsc.VectorSubcoreMesh(core_axis_name="c", subcore_axis_name="s")


def _swish(x):
    return x / (1.0 + jnp.exp(-x))


@functools.partial(
    pl.kernel, mesh=_mesh(),
    compiler_params=pltpu.CompilerParams(needs_layout_passes=False),
    out_type=[
        jax.ShapeDtypeStruct((EP,), jnp.int32),
        jax.ShapeDtypeStruct((EP,), jnp.int32),
        jax.ShapeDtypeStruct((EP,), jnp.float32),
        jax.ShapeDtypeStruct((EP,), jnp.float32),
        jax.ShapeDtypeStruct((EP,), jnp.float32),
    ],
    scratch_types=[
        pltpu.VMEM((NAP,), jnp.int32),
        pltpu.VMEM((3 * NAP,), jnp.float32),
        pltpu.VMEM((128,), jnp.int32),
        pltpu.VMEM((128,), jnp.int32),
        pltpu.VMEM((128,), jnp.int32),
        pltpu.VMEM((128,), jnp.int32),
        pltpu.VMEM((128,), jnp.int32),
        pltpu.VMEM((128,), jnp.float32),
        pltpu.VMEM((128,), jnp.float32),
        pltpu.VMEM((128,), jnp.float32),
    ],
)
def _sc_geom(z_hbm, r_hbm, ii_hbm, ij_hbm, t3i_hbm, t3j_hbm, t3k_hbm,
             zi_hbm, zj_hbm, d2_hbm, xa_hbm, yc_hbm,
             z_v, r_v, ia_v, ib_v, ic_v, oza_v, ozb_v, ofa_v, ofb_v, ofc_v):
    cid = lax.axis_index("c")
    sid = lax.axis_index("s")
    wid = sid * NC + cid
    base = wid * PER_W
    pltpu.sync_copy(z_hbm, z_v)
    pltpu.sync_copy(r_hbm, r_v)

    def echunk(c, _):
        off = base + c * 128
        pltpu.sync_copy(ii_hbm.at[pl.ds(off, 128)], ia_v)
        pltpu.sync_copy(ij_hbm.at[pl.ds(off, 128)], ib_v)

        def vec(j, _):
            s = j * L
            vi = ia_v[pl.ds(s, L)]
            vj = ib_v[pl.ds(s, L)]
            oza_v[pl.ds(s, L)] = plsc.load_gather(z_v, [vi])
            ozb_v[pl.ds(s, L)] = plsc.load_gather(z_v, [vj])
            xi = plsc.load_gather(r_v, [vi * 3])
            yi = plsc.load_gather(r_v, [vi * 3 + 1])
            zi = plsc.load_gather(r_v, [vi * 3 + 2])
            xj = plsc.load_gather(r_v, [vj * 3])
            yj = plsc.load_gather(r_v, [vj * 3 + 1])
            zj = plsc.load_gather(r_v, [vj * 3 + 2])
            dx, dy, dz = xi - xj, yi - yj, zi - zj
            ofa_v[pl.ds(s, L)] = dx * dx + dy * dy + dz * dz
            return _

        lax.fori_loop(0, 128 // L, vec, 0)
        pltpu.sync_copy(oza_v, zi_hbm.at[pl.ds(off, 128)])
        pltpu.sync_copy(ozb_v, zj_hbm.at[pl.ds(off, 128)])
        pltpu.sync_copy(ofa_v, d2_hbm.at[pl.ds(off, 128)])
        return _

    lax.fori_loop(0, NCHUNK, echunk, 0)

    def tchunk(c, _):
        off = base + c * 128
        pltpu.sync_copy(t3i_hbm.at[pl.ds(off, 128)], ia_v)
        pltpu.sync_copy(t3j_hbm.at[pl.ds(off, 128)], ib_v)
        pltpu.sync_copy(t3k_hbm.at[pl.ds(off, 128)], ic_v)

        def vec(j, _):
            s = j * L
            vi = ia_v[pl.ds(s, L)]
            vj = ib_v[pl.ds(s, L)]
            vk = ic_v[pl.ds(s, L)]
            xi = plsc.load_gather(r_v, [vi * 3])
            yi = plsc.load_gather(r_v, [vi * 3 + 1])
            zi = plsc.load_gather(r_v, [vi * 3 + 2])
            ax = plsc.load_gather(r_v, [vj * 3]) - xi
            ay = plsc.load_gather(r_v, [vj * 3 + 1]) - yi
            az = plsc.load_gather(r_v, [vj * 3 + 2]) - zi
            bx = plsc.load_gather(r_v, [vk * 3]) - xi
            by = plsc.load_gather(r_v, [vk * 3 + 1]) - yi
            bz = plsc.load_gather(r_v, [vk * 3 + 2]) - zi
            ofb_v[pl.ds(s, L)] = ax * bx + ay * by + az * bz
            cx = ay * bz - az * by
            cy = az * bx - ax * bz
            cz = ax * by - ay * bx
            ofc_v[pl.ds(s, L)] = cx * cx + cy * cy + cz * cz
            return _

        lax.fori_loop(0, 128 // L, vec, 0)
        pltpu.sync_copy(ofb_v, xa_hbm.at[pl.ds(off, 128)])
        pltpu.sync_copy(ofc_v, yc_hbm.at[pl.ds(off, 128)])
        return _

    lax.fori_loop(0, NCHUNK, tchunk, 0)


@functools.partial(
    pl.kernel, mesh=_mesh(),
    compiler_params=pltpu.CompilerParams(use_tc_tiling_on_sc=False),
    out_type=jax.ShapeDtypeStruct((EP, F // 2), jnp.int32),
    scratch_types=[
        pltpu.VMEM((128,), jnp.int32),
        pltpu.VMEM((128, F // 2), jnp.int32),
        pltpu.SemaphoreType.DMA,
    ],
)
def _sc_gather128(table_hbm, idx_hbm, out_hbm, idx_v, rows_v, sem):
    wid = lax.axis_index("s") * NC + lax.axis_index("c")
    base = wid * PER_W

    def chunk(c, _):
        off = base + c * 128
        pltpu.sync_copy(idx_hbm.at[pl.ds(off, 128)], idx_v)
        pltpu.async_copy(table_hbm.at[idx_v], rows_v, sem).wait()
        pltpu.sync_copy(rows_v, out_hbm.at[pl.ds(off, 128)])
        return _

    lax.fori_loop(0, NCHUNK, chunk, 0)


@functools.partial(
    pl.kernel, mesh=_mesh(),
    compiler_params=pltpu.CompilerParams(use_tc_tiling_on_sc=False),
    out_type=jax.ShapeDtypeStruct((EP, 16), jnp.float32),
    scratch_types=[
        pltpu.VMEM((128,), jnp.int32),
        pltpu.VMEM((128, 16), jnp.float32),
        pltpu.SemaphoreType.DMA,
    ],
)
def _sc_gather16(table_hbm, idx_hbm, out_hbm, idx_v, rows_v, sem):
    wid = lax.axis_index("s") * NC + lax.axis_index("c")
    base = wid * PER_W

    def chunk(c, _):
        off = base + c * 128
        pltpu.sync_copy(idx_hbm.at[pl.ds(off, 128)], idx_v)
        pltpu.async_copy(table_hbm.at[idx_v], rows_v, sem).wait()
        pltpu.sync_copy(rows_v, out_hbm.at[pl.ds(off, 128)])
        return _

    lax.fori_loop(0, NCHUNK, chunk, 0)


@functools.partial(
    pl.kernel, mesh=_mesh(),
    out_type=jax.ShapeDtypeStruct((NC, NAP, F), jnp.float32),
    scratch_types=[
        pltpu.VMEM((128,), jnp.int32),
        pltpu.VMEM((128, F), jnp.float32),
        pltpu.VMEM_SHARED((NAP, F), jnp.float32),
    ],
)
def _sc_seg_atoms(rows_hbm, idx_hbm, zero_hbm, out_hbm, idx_v, rows_v, acc_s):
    cid = lax.axis_index("c")
    sid = lax.axis_index("s")
    wid = sid * NC + cid
    zrows = NAP // NS // 128

    def zchunk(z, _):
        pltpu.sync_copy(zero_hbm, rows_v)
        pltpu.sync_copy(rows_v, acc_s.at[pl.ds(sid * (NAP // NS) + z * 128, 128)])
        return _

    lax.fori_loop(0, zrows, zchunk, 0)
    plsc.subcore_barrier()
    base = wid * PER_W

    def chunk(c, _):
        off = base + c * 128
        pltpu.sync_copy(idx_hbm.at[pl.ds(off, 128)], idx_v)
        pltpu.sync_copy(rows_hbm.at[pl.ds(off, 128)], rows_v)
        pltpu.sync_copy(rows_v, acc_s.at[idx_v], add=True)
        return _

    lax.fori_loop(0, NCHUNK, chunk, 0)
    plsc.subcore_barrier()

    def wchunk(w, _):
        off = sid * (NAP // NS) + w * 128
        pltpu.sync_copy(acc_s.at[pl.ds(off, 128)], rows_v)
        pltpu.sync_copy(rows_v, out_hbm.at[cid, pl.ds(off, 128)])
        return _

    lax.fori_loop(0, zrows, wchunk, 0)


@functools.partial(
    pl.kernel, mesh=_mesh(),
    compiler_params=pltpu.CompilerParams(use_tc_tiling_on_sc=False),
    out_type=jax.ShapeDtypeStruct((EP, 16), jnp.float32),
    scratch_types=[
        pltpu.VMEM((128,), jnp.int32),
        pltpu.VMEM((128,), jnp.int32),
        pltpu.VMEM((128, 16), jnp.float32),
        pltpu.VMEM_SHARED((ACC_E, 16), jnp.float32),
    ],
)
def _sc_seg_edges(rows_hbm, idx_hbm, zero_hbm, out_hbm, idx_v, idx2_v, rows_v, acc_s):
    cid = lax.axis_index("c")
    sid = lax.axis_index("s")
    zrows = HALF_E // NS // 128

    def zchunk(z, _):
        pltpu.sync_copy(zero_hbm, rows_v)
        pltpu.sync_copy(rows_v, acc_s.at[pl.ds(sid * (HALF_E // NS) + z * 128, 128)])
        return _

    lax.fori_loop(0, zrows, zchunk, 0)
    plsc.subcore_barrier()
    base = sid * (EP // NS)
    lo = cid * HALF_E

    def chunk(c, _):
        off = base + c * 128
        pltpu.sync_copy(idx_hbm.at[pl.ds(off, 128)], idx_v)
        pltpu.sync_copy(rows_hbm.at[pl.ds(off, 128)], rows_v)

        def vec(j, _):
            s = j * L
            x = idx_v[pl.ds(s, L)] - lo
            ok = (x >= 0) & (x < HALF_E)
            idx2_v[pl.ds(s, L)] = jnp.where(ok, x, HALF_E)
            return _

        lax.fori_loop(0, 128 // L, vec, 0)
        pltpu.sync_copy(rows_v, acc_s.at[idx2_v], add=True)
        return _

    lax.fori_loop(0, EP // NS // 128, chunk, 0)
    plsc.subcore_barrier()
    wrows = HALF_E // NS // 128

    def wchunk(w, _):
        off = sid * (HALF_E // NS) + w * 128
        pltpu.sync_copy(acc_s.at[pl.ds(off, 128)], rows_v)
        pltpu.sync_copy(rows_v, out_hbm.at[pl.ds(cid * HALF_E + off, 128)])
        return _

    lax.fori_loop(0, wrows, wchunk, 0)


def _tc_feat_body(d2_ref, xa_ref, yc_ref, rbf_ref, rad_ref, ang_ref):
    d2 = d2_ref[...]
    d = jnp.sqrt(jnp.maximum(d2, 0.0) + 1e-12)
    x = d / CUTOFF
    x2 = x * x
    x6 = x2 * x2 * x2
    env = jnp.where(x < 1.0, 1.0 - 28.0 * x6 + 48.0 * x6 * x - 21.0 * x6 * x2, 0.0)
    inv = env / (d + 1e-9)
    n8 = jnp.arange(1, 9, dtype=jnp.int32).astype(jnp.float32)
    sj = jnp.sin(x[:, None] * (n8[None, :] * np.pi))
    rbf = jnp.float32(np.sqrt(2.0 / CUTOFF)) * inv[:, None] * sj
    mask6 = (jnp.arange(8) < N_RBF).astype(jnp.float32)
    rbf_ref[...] = rbf * mask6[None, :]
    n16 = jnp.arange(1, 17, dtype=jnp.int32).astype(jnp.float32)
    s16 = jnp.sin(x[:, None] * (n16[None, :] * np.pi))
    mask7 = (jnp.arange(16) < N_SRAD).astype(jnp.float32)
    rad_ref[...] = inv[:, None] * s16 * mask7[None, :]
    xa = xa_ref[...]
    yc = yc_ref[...]
    r = jnp.sqrt(xa * xa + yc + 1e-12)
    cth = xa / jnp.maximum(r, 1e-30)
    t0 = jnp.ones_like(cth)
    ts = [t0, cth]
    for _ in range(N_SPH - 2):
        ts.append(2.0 * cth * ts[-1] - ts[-2])
    ang = jnp.stack(ts + [jnp.zeros_like(cth)], axis=1)
    ang_ref[...] = ang


def _tc_feat(d2, xa, yc):
    return pl.pallas_call(
        _tc_feat_body,
        grid=(NBE,),
        in_specs=[pl.BlockSpec((BE,), lambda i: (i,))] * 3,
        out_specs=[
            pl.BlockSpec((BE, 8), lambda i: (i, 0)),
            pl.BlockSpec((BE, 16), lambda i: (i, 0)),
            pl.BlockSpec((BE, 8), lambda i: (i, 0)),
        ],
        out_shape=[
            jax.ShapeDtypeStruct((EP, 8), jnp.float32),
            jax.ShapeDtypeStruct((EP, 16), jnp.float32),
            jax.ShapeDtypeStruct((EP, 8), jnp.float32),
        ],
    )(d2, xa, yc)


def _tc_embed_body(zi_ref, zj_ref, rbf_ref, emb_ref, wemb_ref, wrbf_ref,
                   bemb_ref, wout_ref, m_ref, t_ref):
    emb = emb_ref[...]
    w = wemb_ref[...]
    ej_t = jnp.dot(emb, w[0:F, :], preferred_element_type=jnp.float32)
    ei_t = jnp.dot(emb, w[F:2 * F, :], preferred_element_type=jnp.float32)
    w4 = jnp.dot(wrbf_ref[...], w[2 * F:3 * F, :], preferred_element_type=jnp.float32)
    ids = jnp.arange(128, dtype=jnp.int32)
    oh_j = (zj_ref[...][:, None] == ids[None, :]).astype(jnp.float32)
    oh_i = (zi_ref[...][:, None] == ids[None, :]).astype(jnp.float32)
    acc = jnp.dot(oh_j, ej_t, preferred_element_type=jnp.float32)
    acc += jnp.dot(oh_i, ei_t, preferred_element_type=jnp.float32)
    acc += jnp.dot(rbf_ref[...], w4, preferred_element_type=jnp.float32)
    m = _swish(acc + bemb_ref[...])
    m_ref[...] = m
    t_ref[...] = jnp.dot(rbf_ref[...], wout_ref[...],
                         preferred_element_type=jnp.float32) * m


def _tc_embed(zi, zj, rbf8, embP, W_emb, W_emb_rbf8, b_emb, Wrbf_out8_0):
    return pl.pallas_call(
        _tc_embed_body,
        grid=(NBE,),
        in_specs=[
            pl.BlockSpec((BE,), lambda i: (i,)),
            pl.BlockSpec((BE,), lambda i: (i,)),
            pl.BlockSpec((BE, 8), lambda i: (i, 0)),
            pl.BlockSpec((128, F), lambda i: (0, 0)),
            pl.BlockSpec((3 * F, F), lambda i: (0, 0)),
            pl.BlockSpec((8, F), lambda i: (0, 0)),
            pl.BlockSpec((1, F), lambda i: (0, 0)),
            pl.BlockSpec((8, F), lambda i: (0, 0)),
        ],
        out_specs=[
            pl.BlockSpec((BE, F), lambda i: (i, 0)),
            pl.BlockSpec((BE, F), lambda i: (i, 0)),
        ],
        out_shape=[
            jax.ShapeDtypeStruct((EP, F), jnp.float32),
            jax.ShapeDtypeStruct((EP, F), jnp.float32),
        ],
    )(zi, zj, rbf8, embP, W_emb, W_emb_rbf8, b_emb, Wrbf_out8_0)


def _tc_int1_body(m_ref, rbf_ref, wji_ref, bji_ref, wkj_ref, bkj_ref, wri_ref,
                  xji_ref, xkj_ref):
    m = m_ref[...]
    xji_ref[...] = _swish(jnp.dot(m, wji_ref[...], preferred_element_type=jnp.float32)
                          + bji_ref[...])
    xkj = _swish(jnp.dot(m, wkj_ref[...], preferred_element_type=jnp.float32)
                 + bkj_ref[...])
    x = xkj * jnp.dot(rbf_ref[...], wri_ref[...],
                      preferred_element_type=jnp.float32)
    bl = jax.lax.bitcast_convert_type(x[:, 0:64], jnp.int32) + 0x8000
    bh = jax.lax.bitcast_convert_type(x[:, 64:128], jnp.int32) + 0x8000
    xkj_ref[...] = ((bl >> 16) & 0xFFFF) | (bh & ~0xFFFF)


def _tc_int1(m, rbf8, Wji_i, bji_i, Wkj_i, bkj_i, Wrbf_int8_i):
    return pl.pallas_call(
        _tc_int1_body,
        grid=(NBE,),
        in_specs=[
            pl.BlockSpec((BE, F), lambda i: (i, 0)),
            pl.BlockSpec((BE, 8), lambda i: (i, 0)),
            pl.BlockSpec((F, F), lambda i: (0, 0)),
            pl.BlockSpec((1, F), lambda i: (0, 0)),
            pl.BlockSpec((F, F), lambda i: (0, 0)),
            pl.BlockSpec((1, F), lambda i: (0, 0)),
            pl.BlockSpec((8, F), lambda i: (0, 0)),
        ],
        out_specs=[
            pl.BlockSpec((BE, F), lambda i: (i, 0)),
            pl.BlockSpec((BE, F // 2), lambda i: (i, 0)),
        ],
        out_shape=[
            jax.ShapeDtypeStruct((EP, F), jnp.float32),
            jax.ShapeDtypeStruct((EP, F // 2), jnp.int32),
        ],
    )(m, rbf8, Wji_i, bji_i, Wkj_i, bkj_i, Wrbf_int8_i)


def _tc_trip_body(tkj_ref, ang_ref, rad_ref, wl_ref, xb_ref):
    u = tkj_ref[...]
    f_lo = jax.lax.bitcast_convert_type(u << 16, jnp.float32)
    f_hi = jax.lax.bitcast_convert_type(u & ~0xFFFF, jnp.float32)
    tkj = jnp.concatenate([f_lo, f_hi], axis=1)
    ang = ang_ref[...]
    rad = rad_ref[...][:, 0:8]
    sbf = jnp.concatenate([ang[:, l:l + 1] * rad for l in range(N_SPH)], axis=1)
    cols = []
    for b in range(N_BIL):
        tb = jnp.dot(tkj, wl_ref[b], preferred_element_type=jnp.float32)
        cols.append(jnp.sum(tb * sbf, axis=1, keepdims=True))
    xb = jnp.concatenate(cols, axis=1)
    xb_ref[...] = jnp.concatenate([xb, jnp.zeros_like(xb)], axis=1)


def _tc_trip(tkj, ang8, rad16, Wl_i):
    return pl.pallas_call(
        _tc_trip_body,
        grid=(NBE,),
        in_specs=[
            pl.BlockSpec((BE, F // 2), lambda i: (i, 0)),
            pl.BlockSpec((BE, 8), lambda i: (i, 0)),
            pl.BlockSpec((BE, 16), lambda i: (i, 0)),
            pl.BlockSpec((N_BIL, F, 56), lambda i: (0, 0, 0)),
        ],
        out_specs=pl.BlockSpec((BE, 16), lambda i: (i, 0)),
        out_shape=jax.ShapeDtypeStruct((EP, 16), jnp.float32),
    )(tkj, ang8, rad16, Wl_i)


def _tc_res_body(m_ref, xji_ref, xbs_ref, rbf_ref, wbo_ref, bbo_ref,
                 wres_ref, bres_ref, wout_ref, mnew_ref, t_ref):
    xb = _swish(jnp.dot(xbs_ref[...], wbo_ref[...],
                        preferred_element_type=jnp.float32) + bbo_ref[...])
    h = _swish(jnp.dot(xji_ref[...] + xb, wres_ref[...],
                       preferred_element_type=jnp.float32) + bres_ref[...])
    mnew = m_ref[...] + h
    mnew_ref[...] = mnew
    t_ref[...] = jnp.dot(rbf_ref[...], wout_ref[...],
                         preferred_element_type=jnp.float32) * mnew


def _tc_res(m, xji, xbsum16, rbf8, Wbilo16_i, bbilo_i, Wres_i, bres_i, Wrbf_out8_n):
    return pl.pallas_call(
        _tc_res_body,
        grid=(NBE,),
        in_specs=[
            pl.BlockSpec((BE, F), lambda i: (i, 0)),
            pl.BlockSpec((BE, F), lambda i: (i, 0)),
            pl.BlockSpec((BE, 16), lambda i: (i, 0)),
            pl.BlockSpec((BE, 8), lambda i: (i, 0)),
            pl.BlockSpec((16, F), lambda i: (0, 0)),
            pl.BlockSpec((1, F), lambda i: (0, 0)),
            pl.BlockSpec((F, F), lambda i: (0, 0)),
            pl.BlockSpec((1, F), lambda i: (0, 0)),
            pl.BlockSpec((8, F), lambda i: (0, 0)),
        ],
        out_specs=[
            pl.BlockSpec((BE, F), lambda i: (i, 0)),
            pl.BlockSpec((BE, F), lambda i: (i, 0)),
        ],
        out_shape=[
            jax.ShapeDtypeStruct((EP, F), jnp.float32),
            jax.ShapeDtypeStruct((EP, F), jnp.float32),
        ],
    )(m, xji, xbsum16, rbf8, Wbilo16_i, bbilo_i, Wres_i, bres_i, Wrbf_out8_n)


def _tc_out_body(a_ref, w1_ref, b1_ref, w2_ref, p_ref, pnew_ref):
    a = a_ref[0] + a_ref[1]
    a = _swish(jnp.dot(a, w1_ref[...], preferred_element_type=jnp.float32)
               + b1_ref[...])
    pnew_ref[...] = p_ref[...] + jnp.dot(a, w2_ref[...],
                                         preferred_element_type=jnp.float32)


def _tc_out(a_parts, W1_k, b1_k, W2_k, P_prev):
    return pl.pallas_call(
        _tc_out_body,
        grid=(NBA,),
        in_specs=[
            pl.BlockSpec((NC, BA, F), lambda i: (0, i, 0)),
            pl.BlockSpec((F, F), lambda i: (0, 0)),
            pl.BlockSpec((1, F), lambda i: (0, 0)),
            pl.BlockSpec((F, F), lambda i: (0, 0)),
            pl.BlockSpec((BA, F), lambda i: (i, 0)),
        ],
        out_specs=pl.BlockSpec((BA, F), lambda i: (i, 0)),
        out_shape=jax.ShapeDtypeStruct((NAP, F), jnp.float32),
    )(a_parts, W1_k, b1_k, W2_k, P_prev)


def _tc_mol_body(bs_ref, p_ref, out_ref):
    @pl.when(pl.program_id(0) == 0)
    def _():
        out_ref[...] = jnp.zeros_like(out_ref)

    ids = jnp.arange(NMOLP, dtype=jnp.int32)
    oh = (bs_ref[...][:, None] == ids[None, :]).astype(jnp.float32)
    out_ref[...] += jnp.dot(oh.T, p_ref[...], preferred_element_type=jnp.float32)


def _tc_mol(batch_segP, P):
    return pl.pallas_call(
        _tc_mol_body,
        grid=(NBA,),
        in_specs=[
            pl.BlockSpec((BA,), lambda i: (i,)),
            pl.BlockSpec((BA, F), lambda i: (i, 0)),
        ],
        out_specs=pl.BlockSpec((NMOLP, F), lambda i: (0, 0)),
        out_shape=jax.ShapeDtypeStruct((NMOLP, F), jnp.float32),
    )(batch_segP, P)


def kernel(Z, R, batch_seg, idnb_i, idnb_j, id_expand_kj, id_reduce_ji,
           id3dnb_i, id3dnb_j, id3dnb_k, emb, W_emb_rbf, W_emb, b_emb,
           Wrbf_out, W1_out, b1_out, W2_out, Wrbf_int, Wbil, Wji, bji,
           Wkj, bkj, Wbilo, bbilo, Wres, bres):
    i32 = jnp.int32
    pe = EP - N_EDGES
    ii = jnp.concatenate([idnb_i.astype(i32), jnp.full((pe,), N_ATOMS, i32)])
    ij = jnp.concatenate([idnb_j.astype(i32), jnp.full((pe,), N_ATOMS, i32)])
    t3i = jnp.concatenate([id3dnb_i.astype(i32), jnp.zeros((pe,), i32)])
    t3j = jnp.concatenate([id3dnb_j.astype(i32), jnp.zeros((pe,), i32)])
    t3k = jnp.concatenate([id3dnb_k.astype(i32), jnp.zeros((pe,), i32)])
    iexp = jnp.concatenate([id_expand_kj.astype(i32), jnp.zeros((pe,), i32)])
    ired = jnp.concatenate([id_reduce_ji.astype(i32), jnp.full((pe,), EP - 8, i32)])
    ZP = jnp.concatenate([Z.astype(i32), jnp.zeros((NAP - N_ATOMS,), i32)])
    RP = jnp.concatenate([R, jnp.zeros((NAP - N_ATOMS, 3), jnp.float32)]).reshape(-1)
    bsP = jnp.concatenate([batch_seg.astype(i32), jnp.full((NAP - N_ATOMS,), N_MOL, i32)])
    embP = jnp.concatenate([emb, jnp.zeros((128 - emb.shape[0], F), jnp.float32)])
    Wrbf8 = jnp.concatenate([W_emb_rbf, jnp.zeros((2, F), jnp.float32)])
    Wrbf_out8 = jnp.concatenate([Wrbf_out, jnp.zeros((8, 2, F), jnp.float32)], axis=1)
    Wrbf_int8 = jnp.concatenate([Wrbf_int, jnp.zeros((N_INT, 2, F), jnp.float32)], axis=1)
    Wl = jnp.transpose(Wbil, (0, 1, 3, 2)).reshape(N_INT, N_BIL, F, N_SPH, N_SRAD)
    Wl = jnp.concatenate([Wl, jnp.zeros((N_INT, N_BIL, F, N_SPH, 1), jnp.float32)],
                         axis=4).reshape(N_INT, N_BIL, F, 56)
    Wbilo16 = jnp.concatenate([Wbilo, jnp.zeros((N_INT, 8, F), jnp.float32)], axis=1)
    b_emb2 = b_emb.reshape(1, F)
    zero128 = jnp.zeros((128, F), jnp.float32)
    zero16 = jnp.zeros((128, 16), jnp.float32)

    zi, zj, d2, xang, yc2 = _sc_geom(ZP, RP, ii, ij, t3i, t3j, t3k)
    rbf8, rad16, ang8 = _tc_feat(d2, xang, yc2)
    rad_t = _sc_gather16(rad16, iexp)
    m, t = _tc_embed(zi, zj, rbf8, embP, W_emb, Wrbf8, b_emb2, Wrbf_out8[0])
    a_parts = _sc_seg_atoms(t, ii, zero128)
    P = _tc_out(a_parts, W1_out[0], b1_out[0].reshape(1, F), W2_out[0],
                jnp.zeros((NAP, F), jnp.float32))
    for it in range(N_INT):
        xji, xkj = _tc_int1(m, rbf8, Wji[it], bji[it].reshape(1, F),
                            Wkj[it], bkj[it].reshape(1, F), Wrbf_int8[it])
        tkj = _sc_gather128(xkj, iexp)
        xb16 = _tc_trip(tkj, ang8, rad_t, Wl[it])
        xbsum = _sc_seg_edges(xb16, ired, zero16)
        m, t = _tc_res(m, xji, xbsum, rbf8, Wbilo16[it],
                       bbilo[it].reshape(1, F), Wres[it],
                       bres[it].reshape(1, F), Wrbf_out8[it + 1])
        a_parts = _sc_seg_atoms(t, ii, zero128)
        P = _tc_out(a_parts, W1_out[it + 1], b1_out[it + 1].reshape(1, F),
                    W2_out[it + 1], P)
    out = _tc_mol(bsP, P)
    return out[:N_MOL]


# speedup vs baseline: 1.0183x; 1.0183x over previous
import functools
import jax
import jax.numpy as jnp
import numpy as np
from jax import lax
from jax.experimental import pallas as pl
from jax.experimental.pallas import tpu as pltpu
from jax.experimental.pallas import tpu_sc as plsc

N_ATOMS = 10000
N_EDGES = 160000
N_TRIP = 160000
F = 128
N_RBF = 6
N_SPH = 7
N_SRAD = 7
N_BIL = 8
CUTOFF = 5.0
N_MOL = 100
N_INT = 7

NC, NS, L = 2, 16, 16
NW = NC * NS
EP = 163840
PER_W = EP // NW
NCHUNK = PER_W // 128
NAP = 10240
HALF_E = EP // 2
ACC_E = HALF_E + 128
BE = 1024
NBE = EP // BE
BA = 1024
NBA = NAP // BA
NMOLP = 104

_mesh = lambda: plsc.VectorSubcoreMesh(core_axis_name="c", subcore_axis_name="s")


def _swish(x):
    return x / (1.0 + jnp.exp(-x))


@functools.partial(
    pl.kernel, mesh=_mesh(),
    compiler_params=pltpu.CompilerParams(needs_layout_passes=False),
    out_type=[
        jax.ShapeDtypeStruct((EP,), jnp.int32),
        jax.ShapeDtypeStruct((EP,), jnp.int32),
        jax.ShapeDtypeStruct((EP,), jnp.float32),
        jax.ShapeDtypeStruct((EP,), jnp.float32),
        jax.ShapeDtypeStruct((EP,), jnp.float32),
    ],
    scratch_types=[
        pltpu.VMEM((NAP,), jnp.int32),
        pltpu.VMEM((3 * NAP,), jnp.float32),
        pltpu.VMEM((128,), jnp.int32),
        pltpu.VMEM((128,), jnp.int32),
        pltpu.VMEM((128,), jnp.int32),
        pltpu.VMEM((128,), jnp.int32),
        pltpu.VMEM((128,), jnp.int32),
        pltpu.VMEM((128,), jnp.float32),
        pltpu.VMEM((128,), jnp.float32),
        pltpu.VMEM((128,), jnp.float32),
    ],
)
def _sc_geom(z_hbm, r_hbm, ii_hbm, ij_hbm, t3i_hbm, t3j_hbm, t3k_hbm,
             zi_hbm, zj_hbm, d2_hbm, xa_hbm, yc_hbm,
             z_v, r_v, ia_v, ib_v, ic_v, oza_v, ozb_v, ofa_v, ofb_v, ofc_v):
    cid = lax.axis_index("c")
    sid = lax.axis_index("s")
    wid = sid * NC + cid
    base = wid * PER_W
    pltpu.sync_copy(z_hbm, z_v)
    pltpu.sync_copy(r_hbm, r_v)

    def echunk(c, _):
        off = base + c * 128
        pltpu.sync_copy(ii_hbm.at[pl.ds(off, 128)], ia_v)
        pltpu.sync_copy(ij_hbm.at[pl.ds(off, 128)], ib_v)

        def vec(j, _):
            s = j * L
            vi = ia_v[pl.ds(s, L)]
            vj = ib_v[pl.ds(s, L)]
            oza_v[pl.ds(s, L)] = plsc.load_gather(z_v, [vi])
            ozb_v[pl.ds(s, L)] = plsc.load_gather(z_v, [vj])
            xi = plsc.load_gather(r_v, [vi * 3])
            yi = plsc.load_gather(r_v, [vi * 3 + 1])
            zi = plsc.load_gather(r_v, [vi * 3 + 2])
            xj = plsc.load_gather(r_v, [vj * 3])
            yj = plsc.load_gather(r_v, [vj * 3 + 1])
            zj = plsc.load_gather(r_v, [vj * 3 + 2])
            dx, dy, dz = xi - xj, yi - yj, zi - zj
            ofa_v[pl.ds(s, L)] = dx * dx + dy * dy + dz * dz
            return _

        lax.fori_loop(0, 128 // L, vec, 0)
        pltpu.sync_copy(oza_v, zi_hbm.at[pl.ds(off, 128)])
        pltpu.sync_copy(ozb_v, zj_hbm.at[pl.ds(off, 128)])
        pltpu.sync_copy(ofa_v, d2_hbm.at[pl.ds(off, 128)])
        return _

    lax.fori_loop(0, NCHUNK, echunk, 0)

    def tchunk(c, _):
        off = base + c * 128
        pltpu.sync_copy(t3i_hbm.at[pl.ds(off, 128)], ia_v)
        pltpu.sync_copy(t3j_hbm.at[pl.ds(off, 128)], ib_v)
        pltpu.sync_copy(t3k_hbm.at[pl.ds(off, 128)], ic_v)

        def vec(j, _):
            s = j * L
            vi = ia_v[pl.ds(s, L)]
            vj = ib_v[pl.ds(s, L)]
            vk = ic_v[pl.ds(s, L)]
            xi = plsc.load_gather(r_v, [vi * 3])
            yi = plsc.load_gather(r_v, [vi * 3 + 1])
            zi = plsc.load_gather(r_v, [vi * 3 + 2])
            ax = plsc.load_gather(r_v, [vj * 3]) - xi
            ay = plsc.load_gather(r_v, [vj * 3 + 1]) - yi
            az = plsc.load_gather(r_v, [vj * 3 + 2]) - zi
            bx = plsc.load_gather(r_v, [vk * 3]) - xi
            by = plsc.load_gather(r_v, [vk * 3 + 1]) - yi
            bz = plsc.load_gather(r_v, [vk * 3 + 2]) - zi
            ofb_v[pl.ds(s, L)] = ax * bx + ay * by + az * bz
            cx = ay * bz - az * by
            cy = az * bx - ax * bz
            cz = ax * by - ay * bx
            ofc_v[pl.ds(s, L)] = cx * cx + cy * cy + cz * cz
            return _

        lax.fori_loop(0, 128 // L, vec, 0)
        pltpu.sync_copy(ofb_v, xa_hbm.at[pl.ds(off, 128)])
        pltpu.sync_copy(ofc_v, yc_hbm.at[pl.ds(off, 128)])
        return _

    lax.fori_loop(0, NCHUNK, tchunk, 0)


@functools.partial(
    pl.kernel, mesh=_mesh(),
    compiler_params=pltpu.CompilerParams(use_tc_tiling_on_sc=False),
    out_type=jax.ShapeDtypeStruct((EP, F // 2), jnp.int32),
    scratch_types=[
        pltpu.VMEM((128,), jnp.int32),
        pltpu.VMEM((128,), jnp.int32),
        pltpu.VMEM((128, F // 2), jnp.int32),
        pltpu.VMEM((128, F // 2), jnp.int32),
        pltpu.SemaphoreType.DMA,
        pltpu.SemaphoreType.DMA,
        pltpu.SemaphoreType.DMA,
        pltpu.SemaphoreType.DMA,
    ],
)
def _sc_gather128(table_hbm, idx_hbm, out_hbm, ia_v, ib_v, ra_v, rb_v,
                  s0, s1, s2, s3):
    wid = lax.axis_index("s") * NC + lax.axis_index("c")
    base = wid * PER_W

    def chunk2(c, _):
        off0 = base + (2 * c) * 128
        off1 = off0 + 128
        pltpu.sync_copy(idx_hbm.at[pl.ds(off0, 128)], ia_v)
        g0 = pltpu.async_copy(table_hbm.at[ia_v], ra_v, s0)
        pltpu.sync_copy(idx_hbm.at[pl.ds(off1, 128)], ib_v)
        g1 = pltpu.async_copy(table_hbm.at[ib_v], rb_v, s1)
        g0.wait()
        w0 = pltpu.async_copy(ra_v, out_hbm.at[pl.ds(off0, 128)], s2)
        g1.wait()
        w1 = pltpu.async_copy(rb_v, out_hbm.at[pl.ds(off1, 128)], s3)
        w0.wait()
        w1.wait()
        return _

    lax.fori_loop(0, NCHUNK // 2, chunk2, 0)


@functools.partial(
    pl.kernel, mesh=_mesh(),
    compiler_params=pltpu.CompilerParams(use_tc_tiling_on_sc=False),
    out_type=jax.ShapeDtypeStruct((EP, 16), jnp.float32),
    scratch_types=[
        pltpu.VMEM((128,), jnp.int32),
        pltpu.VMEM((128, 16), jnp.float32),
        pltpu.SemaphoreType.DMA,
    ],
)
def _sc_gather16(table_hbm, idx_hbm, out_hbm, idx_v, rows_v, sem):
    wid = lax.axis_index("s") * NC + lax.axis_index("c")
    base = wid * PER_W

    def chunk(c, _):
        off = base + c * 128
        pltpu.sync_copy(idx_hbm.at[pl.ds(off, 128)], idx_v)
        pltpu.async_copy(table_hbm.at[idx_v], rows_v, sem).wait()
        pltpu.sync_copy(rows_v, out_hbm.at[pl.ds(off, 128)])
        return _

    lax.fori_loop(0, NCHUNK, chunk, 0)


@functools.partial(
    pl.kernel, mesh=_mesh(),
    out_type=jax.ShapeDtypeStruct((NC, NAP, F), jnp.float32),
    scratch_types=[
        pltpu.VMEM((128,), jnp.int32),
        pltpu.VMEM((128, F), jnp.float32),
        pltpu.VMEM_SHARED((NAP, F), jnp.float32),
    ],
)
def _sc_seg_atoms(rows_hbm, idx_hbm, zero_hbm, out_hbm, idx_v, rows_v, acc_s):
    cid = lax.axis_index("c")
    sid = lax.axis_index("s")
    wid = sid * NC + cid
    zrows = NAP // NS // 128

    def zchunk(z, _):
        pltpu.sync_copy(zero_hbm, rows_v)
        pltpu.sync_copy(rows_v, acc_s.at[pl.ds(sid * (NAP // NS) + z * 128, 128)])
        return _

    lax.fori_loop(0, zrows, zchunk, 0)
    plsc.subcore_barrier()
    base = wid * PER_W

    def chunk(c, _):
        off = base + c * 128
        pltpu.sync_copy(idx_hbm.at[pl.ds(off, 128)], idx_v)
        pltpu.sync_copy(rows_hbm.at[pl.ds(off, 128)], rows_v)
        pltpu.sync_copy(rows_v, acc_s.at[idx_v], add=True)
        return _

    lax.fori_loop(0, NCHUNK, chunk, 0)
    plsc.subcore_barrier()

    def wchunk(w, _):
        off = sid * (NAP // NS) + w * 128
        pltpu.sync_copy(acc_s.at[pl.ds(off, 128)], rows_v)
        pltpu.sync_copy(rows_v, out_hbm.at[cid, pl.ds(off, 128)])
        return _

    lax.fori_loop(0, zrows, wchunk, 0)


@functools.partial(
    pl.kernel, mesh=_mesh(),
    compiler_params=pltpu.CompilerParams(use_tc_tiling_on_sc=False),
    out_type=jax.ShapeDtypeStruct((EP, 16), jnp.float32),
    scratch_types=[
        pltpu.VMEM((128,), jnp.int32),
        pltpu.VMEM((128,), jnp.int32),
        pltpu.VMEM((128,), jnp.int32),
        pltpu.VMEM((128, 16), jnp.float32),
        pltpu.VMEM((128, 16), jnp.float32),
        pltpu.SemaphoreType.DMA,
        pltpu.SemaphoreType.DMA,
        pltpu.VMEM_SHARED((ACC_E, 16), jnp.float32),
    ],
)
def _sc_seg_edges(rows_hbm, idx_hbm, zero_hbm, out_hbm, idx_v, idxb_v, idx2_v,
                  rows_v, rowsb_v, s0, s1, acc_s):
    cid = lax.axis_index("c")
    sid = lax.axis_index("s")
    zrows = HALF_E // NS // 128

    def zchunk(z, _):
        pltpu.sync_copy(zero_hbm, rows_v)
        pltpu.sync_copy(rows_v, acc_s.at[pl.ds(sid * (HALF_E // NS) + z * 128, 128)])
        return _

    lax.fori_loop(0, zrows, zchunk, 0)
    plsc.subcore_barrier()
    base = sid * (EP // NS)
    lo = cid * HALF_E

    def remap(src_v, _):
        def vec(j, _):
            s = j * L
            x = src_v[pl.ds(s, L)] - lo
            ok = (x >= 0) & (x < HALF_E)
            idx2_v[pl.ds(s, L)] = jnp.where(ok, x, HALF_E)
            return _

        lax.fori_loop(0, 128 // L, vec, 0)

    def chunk2(c, _):
        off0 = base + (2 * c) * 128
        off1 = off0 + 128
        pltpu.sync_copy(idx_hbm.at[pl.ds(off0, 128)], idx_v)
        g0 = pltpu.async_copy(rows_hbm.at[pl.ds(off0, 128)], rows_v, s0)
        pltpu.sync_copy(idx_hbm.at[pl.ds(off1, 128)], idxb_v)
        g1 = pltpu.async_copy(rows_hbm.at[pl.ds(off1, 128)], rowsb_v, s1)
        remap(idx_v, 0)
        g0.wait()
        pltpu.sync_copy(rows_v, acc_s.at[idx2_v], add=True)
        remap(idxb_v, 0)
        g1.wait()
        pltpu.sync_copy(rowsb_v, acc_s.at[idx2_v], add=True)
        return _

    lax.fori_loop(0, EP // NS // 128 // 2, chunk2, 0)
    plsc.subcore_barrier()
    wrows = HALF_E // NS // 128

    def wchunk(w, _):
        off = sid * (HALF_E // NS) + w * 128
        pltpu.sync_copy(acc_s.at[pl.ds(off, 128)], rows_v)
        pltpu.sync_copy(rows_v, out_hbm.at[pl.ds(cid * HALF_E + off, 128)])
        return _

    lax.fori_loop(0, wrows, wchunk, 0)


def _tc_feat_body(d2_ref, xa_ref, yc_ref, rbf_ref, rad_ref, ang_ref):
    d2 = d2_ref[...]
    d = jnp.sqrt(jnp.maximum(d2, 0.0) + 1e-12)
    x = d / CUTOFF
    x2 = x * x
    x6 = x2 * x2 * x2
    env = jnp.where(x < 1.0, 1.0 - 28.0 * x6 + 48.0 * x6 * x - 21.0 * x6 * x2, 0.0)
    inv = env / (d + 1e-9)
    n8 = jnp.arange(1, 9, dtype=jnp.int32).astype(jnp.float32)
    sj = jnp.sin(x[:, None] * (n8[None, :] * np.pi))
    rbf = jnp.float32(np.sqrt(2.0 / CUTOFF)) * inv[:, None] * sj
    mask6 = (jnp.arange(8) < N_RBF).astype(jnp.float32)
    rbf_ref[...] = rbf * mask6[None, :]
    n16 = jnp.arange(1, 17, dtype=jnp.int32).astype(jnp.float32)
    s16 = jnp.sin(x[:, None] * (n16[None, :] * np.pi))
    mask7 = (jnp.arange(16) < N_SRAD).astype(jnp.float32)
    rad_ref[...] = inv[:, None] * s16 * mask7[None, :]
    xa = xa_ref[...]
    yc = yc_ref[...]
    r = jnp.sqrt(xa * xa + yc + 1e-12)
    cth = xa / jnp.maximum(r, 1e-30)
    t0 = jnp.ones_like(cth)
    ts = [t0, cth]
    for _ in range(N_SPH - 2):
        ts.append(2.0 * cth * ts[-1] - ts[-2])
    ang = jnp.stack(ts + [jnp.zeros_like(cth)], axis=1)
    ang_ref[...] = ang


def _tc_feat(d2, xa, yc):
    return pl.pallas_call(
        _tc_feat_body,
        grid=(NBE,),
        in_specs=[pl.BlockSpec((BE,), lambda i: (i,))] * 3,
        out_specs=[
            pl.BlockSpec((BE, 8), lambda i: (i, 0)),
            pl.BlockSpec((BE, 16), lambda i: (i, 0)),
            pl.BlockSpec((BE, 8), lambda i: (i, 0)),
        ],
        out_shape=[
            jax.ShapeDtypeStruct((EP, 8), jnp.float32),
            jax.ShapeDtypeStruct((EP, 16), jnp.float32),
            jax.ShapeDtypeStruct((EP, 8), jnp.float32),
        ],
    )(d2, xa, yc)


def _tc_embed_body(zi_ref, zj_ref, rbf_ref, emb_ref, wemb_ref, wrbf_ref,
                   bemb_ref, wout_ref, m_ref, t_ref):
    emb = emb_ref[...]
    w = wemb_ref[...]
    ej_t = jnp.dot(emb, w[0:F, :], preferred_element_type=jnp.float32)
    ei_t = jnp.dot(emb, w[F:2 * F, :], preferred_element_type=jnp.float32)
    w4 = jnp.dot(wrbf_ref[...], w[2 * F:3 * F, :], preferred_element_type=jnp.float32)
    ids = jnp.arange(128, dtype=jnp.int32)
    oh_j = (zj_ref[...][:, None] == ids[None, :]).astype(jnp.float32)
    oh_i = (zi_ref[...][:, None] == ids[None, :]).astype(jnp.float32)
    acc = jnp.dot(oh_j, ej_t, preferred_element_type=jnp.float32)
    acc += jnp.dot(oh_i, ei_t, preferred_element_type=jnp.float32)
    acc += jnp.dot(rbf_ref[...], w4, preferred_element_type=jnp.float32)
    m = _swish(acc + bemb_ref[...])
    m_ref[...] = m
    t_ref[...] = jnp.dot(rbf_ref[...], wout_ref[...],
                         preferred_element_type=jnp.float32) * m


def _tc_embed(zi, zj, rbf8, embP, W_emb, W_emb_rbf8, b_emb, Wrbf_out8_0):
    return pl.pallas_call(
        _tc_embed_body,
        grid=(NBE,),
        in_specs=[
            pl.BlockSpec((BE,), lambda i: (i,)),
            pl.BlockSpec((BE,), lambda i: (i,)),
            pl.BlockSpec((BE, 8), lambda i: (i, 0)),
            pl.BlockSpec((128, F), lambda i: (0, 0)),
            pl.BlockSpec((3 * F, F), lambda i: (0, 0)),
            pl.BlockSpec((8, F), lambda i: (0, 0)),
            pl.BlockSpec((1, F), lambda i: (0, 0)),
            pl.BlockSpec((8, F), lambda i: (0, 0)),
        ],
        out_specs=[
            pl.BlockSpec((BE, F), lambda i: (i, 0)),
            pl.BlockSpec((BE, F), lambda i: (i, 0)),
        ],
        out_shape=[
            jax.ShapeDtypeStruct((EP, F), jnp.float32),
            jax.ShapeDtypeStruct((EP, F), jnp.float32),
        ],
    )(zi, zj, rbf8, embP, W_emb, W_emb_rbf8, b_emb, Wrbf_out8_0)


def _tc_int1_body(m_ref, rbf_ref, wji_ref, bji_ref, wkj_ref, bkj_ref, wri_ref,
                  xji_ref, xkj_ref):
    m = m_ref[...]
    xji_ref[...] = _swish(jnp.dot(m, wji_ref[...], preferred_element_type=jnp.float32)
                          + bji_ref[...])
    xkj = _swish(jnp.dot(m, wkj_ref[...], preferred_element_type=jnp.float32)
                 + bkj_ref[...])
    x = xkj * jnp.dot(rbf_ref[...], wri_ref[...],
                      preferred_element_type=jnp.float32)
    bl = jax.lax.bitcast_convert_type(x[:, 0:64], jnp.int32) + 0x8000
    bh = jax.lax.bitcast_convert_type(x[:, 64:128], jnp.int32) + 0x8000
    xkj_ref[...] = ((bl >> 16) & 0xFFFF) | (bh & ~0xFFFF)


def _tc_int1(m, rbf8, Wji_i, bji_i, Wkj_i, bkj_i, Wrbf_int8_i):
    return pl.pallas_call(
        _tc_int1_body,
        grid=(NBE,),
        in_specs=[
            pl.BlockSpec((BE, F), lambda i: (i, 0)),
            pl.BlockSpec((BE, 8), lambda i: (i, 0)),
            pl.BlockSpec((F, F), lambda i: (0, 0)),
            pl.BlockSpec((1, F), lambda i: (0, 0)),
            pl.BlockSpec((F, F), lambda i: (0, 0)),
            pl.BlockSpec((1, F), lambda i: (0, 0)),
            pl.BlockSpec((8, F), lambda i: (0, 0)),
        ],
        out_specs=[
            pl.BlockSpec((BE, F), lambda i: (i, 0)),
            pl.BlockSpec((BE, F // 2), lambda i: (i, 0)),
        ],
        out_shape=[
            jax.ShapeDtypeStruct((EP, F), jnp.float32),
            jax.ShapeDtypeStruct((EP, F // 2), jnp.int32),
        ],
    )(m, rbf8, Wji_i, bji_i, Wkj_i, bkj_i, Wrbf_int8_i)


def _tc_trip_body(tkj_ref, ang_ref, rad_ref, wl_ref, xb_ref):
    u = tkj_ref[...]
    f_lo = jax.lax.bitcast_convert_type(u << 16, jnp.float32)
    f_hi = jax.lax.bitcast_convert_type(u & ~0xFFFF, jnp.float32)
    tkj = jnp.concatenate([f_lo, f_hi], axis=1)
    ang = ang_ref[...]
    rad = rad_ref[...][:, 0:8]
    sbf = jnp.concatenate([ang[:, l:l + 1] * rad for l in range(N_SPH)], axis=1)
    cols = []
    for b in range(N_BIL):
        tb = jnp.dot(tkj, wl_ref[b], preferred_element_type=jnp.float32)
        cols.append(jnp.sum(tb * sbf, axis=1, keepdims=True))
    xb = jnp.concatenate(cols, axis=1)
    xb_ref[...] = jnp.concatenate([xb, jnp.zeros_like(xb)], axis=1)


def _tc_trip(tkj, ang8, rad16, Wl_i):
    return pl.pallas_call(
        _tc_trip_body,
        grid=(NBE,),
        in_specs=[
            pl.BlockSpec((BE, F // 2), lambda i: (i, 0)),
            pl.BlockSpec((BE, 8), lambda i: (i, 0)),
            pl.BlockSpec((BE, 16), lambda i: (i, 0)),
            pl.BlockSpec((N_BIL, F, 56), lambda i: (0, 0, 0)),
        ],
        out_specs=pl.BlockSpec((BE, 16), lambda i: (i, 0)),
        out_shape=jax.ShapeDtypeStruct((EP, 16), jnp.float32),
    )(tkj, ang8, rad16, Wl_i)


def _tc_res_body(m_ref, xji_ref, xbs_ref, rbf_ref, wbo_ref, bbo_ref,
                 wres_ref, bres_ref, wout_ref, mnew_ref, t_ref):
    xb = _swish(jnp.dot(xbs_ref[...], wbo_ref[...],
                        preferred_element_type=jnp.float32) + bbo_ref[...])
    h = _swish(jnp.dot(xji_ref[...] + xb, wres_ref[...],
                       preferred_element_type=jnp.float32) + bres_ref[...])
    mnew = m_ref[...] + h
    mnew_ref[...] = mnew
    t_ref[...] = jnp.dot(rbf_ref[...], wout_ref[...],
                         preferred_element_type=jnp.float32) * mnew


def _tc_res(m, xji, xbsum16, rbf8, Wbilo16_i, bbilo_i, Wres_i, bres_i, Wrbf_out8_n):
    return pl.pallas_call(
        _tc_res_body,
        grid=(NBE,),
        in_specs=[
            pl.BlockSpec((BE, F), lambda i: (i, 0)),
            pl.BlockSpec((BE, F), lambda i: (i, 0)),
            pl.BlockSpec((BE, 16), lambda i: (i, 0)),
            pl.BlockSpec((BE, 8), lambda i: (i, 0)),
            pl.BlockSpec((16, F), lambda i: (0, 0)),
            pl.BlockSpec((1, F), lambda i: (0, 0)),
            pl.BlockSpec((F, F), lambda i: (0, 0)),
            pl.BlockSpec((1, F), lambda i: (0, 0)),
            pl.BlockSpec((8, F), lambda i: (0, 0)),
        ],
        out_specs=[
            pl.BlockSpec((BE, F), lambda i: (i, 0)),
            pl.BlockSpec((BE, F), lambda i: (i, 0)),
        ],
        out_shape=[
            jax.ShapeDtypeStruct((EP, F), jnp.float32),
            jax.ShapeDtypeStruct((EP, F), jnp.float32),
        ],
    )(m, xji, xbsum16, rbf8, Wbilo16_i, bbilo_i, Wres_i, bres_i, Wrbf_out8_n)


def _tc_out_body(a_ref, w1_ref, b1_ref, w2_ref, p_ref, pnew_ref):
    a = a_ref[0] + a_ref[1]
    a = _swish(jnp.dot(a, w1_ref[...], preferred_element_type=jnp.float32)
               + b1_ref[...])
    pnew_ref[...] = p_ref[...] + jnp.dot(a, w2_ref[...],
                                         preferred_element_type=jnp.float32)


def _tc_out(a_parts, W1_k, b1_k, W2_k, P_prev):
    return pl.pallas_call(
        _tc_out_body,
        grid=(NBA,),
        in_specs=[
            pl.BlockSpec((NC, BA, F), lambda i: (0, i, 0)),
            pl.BlockSpec((F, F), lambda i: (0, 0)),
            pl.BlockSpec((1, F), lambda i: (0, 0)),
            pl.BlockSpec((F, F), lambda i: (0, 0)),
            pl.BlockSpec((BA, F), lambda i: (i, 0)),
        ],
        out_specs=pl.BlockSpec((BA, F), lambda i: (i, 0)),
        out_shape=jax.ShapeDtypeStruct((NAP, F), jnp.float32),
    )(a_parts, W1_k, b1_k, W2_k, P_prev)


def _tc_mol_body(bs_ref, p_ref, out_ref):
    @pl.when(pl.program_id(0) == 0)
    def _():
        out_ref[...] = jnp.zeros_like(out_ref)

    ids = jnp.arange(NMOLP, dtype=jnp.int32)
    oh = (bs_ref[...][:, None] == ids[None, :]).astype(jnp.float32)
    out_ref[...] += jnp.dot(oh.T, p_ref[...], preferred_element_type=jnp.float32)


def _tc_mol(batch_segP, P):
    return pl.pallas_call(
        _tc_mol_body,
        grid=(NBA,),
        in_specs=[
            pl.BlockSpec((BA,), lambda i: (i,)),
            pl.BlockSpec((BA, F), lambda i: (i, 0)),
        ],
        out_specs=pl.BlockSpec((NMOLP, F), lambda i: (0, 0)),
        out_shape=jax.ShapeDtypeStruct((NMOLP, F), jnp.float32),
    )(batch_segP, P)


def kernel(Z, R, batch_seg, idnb_i, idnb_j, id_expand_kj, id_reduce_ji,
           id3dnb_i, id3dnb_j, id3dnb_k, emb, W_emb_rbf, W_emb, b_emb,
           Wrbf_out, W1_out, b1_out, W2_out, Wrbf_int, Wbil, Wji, bji,
           Wkj, bkj, Wbilo, bbilo, Wres, bres):
    i32 = jnp.int32
    pe = EP - N_EDGES
    ii = jnp.concatenate([idnb_i.astype(i32), jnp.full((pe,), N_ATOMS, i32)])
    ij = jnp.concatenate([idnb_j.astype(i32), jnp.full((pe,), N_ATOMS, i32)])
    t3i = jnp.concatenate([id3dnb_i.astype(i32), jnp.zeros((pe,), i32)])
    t3j = jnp.concatenate([id3dnb_j.astype(i32), jnp.zeros((pe,), i32)])
    t3k = jnp.concatenate([id3dnb_k.astype(i32), jnp.zeros((pe,), i32)])
    iexp = jnp.concatenate([id_expand_kj.astype(i32), jnp.zeros((pe,), i32)])
    ired = jnp.concatenate([id_reduce_ji.astype(i32), jnp.full((pe,), EP - 8, i32)])
    ZP = jnp.concatenate([Z.astype(i32), jnp.zeros((NAP - N_ATOMS,), i32)])
    RP = jnp.concatenate([R, jnp.zeros((NAP - N_ATOMS, 3), jnp.float32)]).reshape(-1)
    bsP = jnp.concatenate([batch_seg.astype(i32), jnp.full((NAP - N_ATOMS,), N_MOL, i32)])
    embP = jnp.concatenate([emb, jnp.zeros((128 - emb.shape[0], F), jnp.float32)])
    Wrbf8 = jnp.concatenate([W_emb_rbf, jnp.zeros((2, F), jnp.float32)])
    Wrbf_out8 = jnp.concatenate([Wrbf_out, jnp.zeros((8, 2, F), jnp.float32)], axis=1)
    Wrbf_int8 = jnp.concatenate([Wrbf_int, jnp.zeros((N_INT, 2, F), jnp.float32)], axis=1)
    Wl = jnp.transpose(Wbil, (0, 1, 3, 2)).reshape(N_INT, N_BIL, F, N_SPH, N_SRAD)
    Wl = jnp.concatenate([Wl, jnp.zeros((N_INT, N_BIL, F, N_SPH, 1), jnp.float32)],
                         axis=4).reshape(N_INT, N_BIL, F, 56)
    Wbilo16 = jnp.concatenate([Wbilo, jnp.zeros((N_INT, 8, F), jnp.float32)], axis=1)
    b_emb2 = b_emb.reshape(1, F)
    zero128 = jnp.zeros((128, F), jnp.float32)
    zero16 = jnp.zeros((128, 16), jnp.float32)

    zi, zj, d2, xang, yc2 = _sc_geom(ZP, RP, ii, ij, t3i, t3j, t3k)
    rbf8, rad16, ang8 = _tc_feat(d2, xang, yc2)
    rad_t = _sc_gather16(rad16, iexp)
    m, t = _tc_embed(zi, zj, rbf8, embP, W_emb, Wrbf8, b_emb2, Wrbf_out8[0])
    a_parts = _sc_seg_atoms(t, ii, zero128)
    P = _tc_out(a_parts, W1_out[0], b1_out[0].reshape(1, F), W2_out[0],
                jnp.zeros((NAP, F), jnp.float32))
    for it in range(N_INT):
        xji, xkj = _tc_int1(m, rbf8, Wji[it], bji[it].reshape(1, F),
                            Wkj[it], bkj[it].reshape(1, F), Wrbf_int8[it])
        tkj = _sc_gather128(xkj, iexp)
        xb16 = _tc_trip(tkj, ang8, rad_t, Wl[it])
        xbsum = _sc_seg_edges(xb16, ired, zero16)
        m, t = _tc_res(m, xji, xbsum, rbf8, Wbilo16[it],
                       bbilo[it].reshape(1, F), Wres[it],
                       bres[it].reshape(1, F), Wrbf_out8[it + 1])
        a_parts = _sc_seg_atoms(t, ii, zero128)
        P = _tc_out(a_parts, W1_out[it + 1], b1_out[it + 1].reshape(1, F),
                    W2_out[it + 1], P)
    out = _tc_mol(bsP, P)
    return out[:N_MOL]
